# Initial kernel scaffold; baseline (speedup 1.0000x reference)
import functools

import jax
import jax.numpy as jnp
from jax import lax
from jax.experimental import pallas as pl
from jax.experimental.pallas import tpu as pltpu
from jax.experimental.pallas import tpu_sc as plsc

N = 10000
E = 320000
H = 128
T = 6

NC = 2
NS = 16
NW = NC * NS
EPW = E // NW
CHUNK = 80
NCHUNK = EPW // CHUNK
RPT = N // NS
ZCH = 125
LANES = 16


def _edge_dense_body(x_ref, w_ref, b_ref, z_ref):
    z_ref[...] = (
        jnp.dot(x_ref[...], w_ref[...], preferred_element_type=jnp.float32)
        + b_ref[...]
    )


def _edge_dense(x, w, b):
    blk = 1000
    return pl.pallas_call(
        _edge_dense_body,
        grid=(N // blk,),
        in_specs=[
            pl.BlockSpec((blk, H), lambda i: (i, 0)),
            pl.BlockSpec((H, T * H), lambda i: (0, 0)),
            pl.BlockSpec((1, T * H), lambda i: (0, 0)),
        ],
        out_specs=pl.BlockSpec((blk, T * H), lambda i: (i, 0)),
        out_shape=jax.ShapeDtypeStruct((N, T * H), jnp.float32),
    )(x, w, b)


_sc_mesh = plsc.VectorSubcoreMesh(core_axis_name="c", subcore_axis_name="s")


@functools.partial(
    pl.kernel,
    out_type=jax.ShapeDtypeStruct((NC, N, H), jnp.float32),
    mesh=_sc_mesh,
    scratch_types=[
        pltpu.VMEM((CHUNK,), jnp.int32),
        pltpu.VMEM((CHUNK,), jnp.int32),
        pltpu.VMEM((CHUNK,), jnp.int32),
        pltpu.VMEM((CHUNK,), jnp.int32),
        pltpu.VMEM((CHUNK, H), jnp.float32),
        pltpu.VMEM((ZCH, H), jnp.float32),
        pltpu.VMEM_SHARED((N, H), jnp.float32),
        pltpu.SemaphoreType.DMA,
    ],
)
def _sc_accumulate(z_hbm, src_hbm, ty_hbm, dst_hbm, out_hbm,
                   src_v, ty_v, dst_v, comb_v, rows_v, zbuf_v, acc_sh, sem):
    cid = lax.axis_index("c")
    sid = lax.axis_index("s")
    wid = cid * NS + sid

    def _zrow(r, carry):
        for c in range(H // LANES):
            zbuf_v[r, pl.ds(c * LANES, LANES)] = jnp.zeros((LANES,), jnp.float32)
        return carry

    lax.fori_loop(0, ZCH, _zrow, 0)
    for k in range(RPT // ZCH):
        pltpu.sync_copy(zbuf_v, acc_sh.at[pl.ds(sid * RPT + k * ZCH, ZCH)])
    plsc.subcore_barrier()

    def _body(i, carry):
        base = wid * EPW + i * CHUNK
        pltpu.sync_copy(src_hbm.at[pl.ds(base, CHUNK)], src_v)
        pltpu.sync_copy(ty_hbm.at[pl.ds(base, CHUNK)], ty_v)
        pltpu.sync_copy(dst_hbm.at[pl.ds(base, CHUNK)], dst_v)
        for j in range(CHUNK // LANES):
            s = pl.ds(j * LANES, LANES)
            comb_v[s] = src_v[s] * T + ty_v[s]
        pltpu.async_copy(z_hbm.at[comb_v], rows_v, sem).wait()
        pltpu.sync_copy(rows_v, acc_sh.at[dst_v], add=True)
        return carry

    lax.fori_loop(0, NCHUNK, _body, 0)
    plsc.subcore_barrier()

    r0 = sid * RPT
    pltpu.sync_copy(acc_sh.at[pl.ds(r0, RPT)], out_hbm.at[cid, pl.ds(r0, RPT)])


def _gru_body(x_ref, p_ref, wi_ref, wh_ref, b_ref, o_ref):
    x = x_ref[...]
    h = p_ref[0] + p_ref[1]
    xg = jnp.dot(x, wi_ref[...], preferred_element_type=jnp.float32) + b_ref[...]
    hg = jnp.dot(h, wh_ref[...], preferred_element_type=jnp.float32)
    r = jax.nn.sigmoid(xg[:, :H] + hg[:, :H])
    z = jax.nn.sigmoid(xg[:, H:2 * H] + hg[:, H:2 * H])
    n = jnp.tanh(xg[:, 2 * H:] + r * hg[:, 2 * H:])
    o_ref[...] = (1.0 - z) * n + z * h


def _gru(x, p, wi, wh, b):
    blk = 1000
    return pl.pallas_call(
        _gru_body,
        grid=(N // blk,),
        in_specs=[
            pl.BlockSpec((blk, H), lambda i: (i, 0)),
            pl.BlockSpec((NC, blk, H), lambda i: (0, i, 0)),
            pl.BlockSpec((H, 3 * H), lambda i: (0, 0)),
            pl.BlockSpec((H, 3 * H), lambda i: (0, 0)),
            pl.BlockSpec((1, 3 * H), lambda i: (0, 0)),
        ],
        out_specs=pl.BlockSpec((blk, H), lambda i: (i, 0)),
        out_shape=jax.ShapeDtypeStruct((N, H), jnp.float32),
    )(x, p, wi, wh, b)


def kernel(statement_embeddings, source_indices, dest_indices, edge_types,
           W_edge, b_edge, Wir, Whr, br, Wiz, Whz, bz, Win, Whn, bn):
    z = _edge_dense(statement_embeddings, W_edge, b_edge.reshape(1, T * H))
    z_rows = z.reshape(N * T, H)
    partials = _sc_accumulate(z_rows, source_indices, edge_types, dest_indices)
    wi = jnp.concatenate([Wir, Wiz, Win], axis=1)
    wh = jnp.concatenate([Whr, Whz, Whn], axis=1)
    b = jnp.concatenate([br, bz, bn]).reshape(1, 3 * H)
    return _gru(statement_embeddings, partials, wi, wh, b)


# speedup vs baseline: 6.6091x; 6.6091x over previous
import functools

import jax
import jax.numpy as jnp
from jax import lax
from jax.experimental import pallas as pl
from jax.experimental.pallas import tpu as pltpu
from jax.experimental.pallas import tpu_sc as plsc

N = 10000
E = 320000
H = 128
T = 6

NC = 2
NS = 16
NW = NC * NS
EPW = E // NW
CHUNK = 80
NCHUNK = EPW // CHUNK
NPAD = 10240
RPT = NPAD // NS
ZCH = 128
LANES = 16


def _edge_dense_body(x_ref, w_ref, b_ref, z_ref):
    z_ref[...] = (
        jnp.dot(x_ref[...], w_ref[...], preferred_element_type=jnp.float32)
        + b_ref[...]
    )


def _edge_dense(x, w, b):
    blk = 1000
    return pl.pallas_call(
        _edge_dense_body,
        grid=(N // blk,),
        in_specs=[
            pl.BlockSpec((blk, H), lambda i: (i, 0)),
            pl.BlockSpec((H, T * H), lambda i: (0, 0)),
            pl.BlockSpec((1, T * H), lambda i: (0, 0)),
        ],
        out_specs=pl.BlockSpec((blk, T * H), lambda i: (i, 0)),
        out_shape=jax.ShapeDtypeStruct((N, T * H), jnp.float32),
    )(x, w, b)


def _sc_accumulate_body(z_hbm, src_hbm, ty_hbm, dst_hbm, out_hbm,
                        src_v, ty_v, dst_v, comb_v, rows_v, zbuf_v, acc_sh,
                        sem):
    cid = lax.axis_index("c")
    sid = lax.axis_index("s")
    wid = cid * NS + sid

    def _zrow(r, carry):
        for c in range(H // LANES):
            zbuf_v[r, pl.ds(c * LANES, LANES)] = jnp.zeros((LANES,), jnp.float32)
        return carry

    lax.fori_loop(0, ZCH, _zrow, 0)
    for k in range(RPT // ZCH):
        pltpu.sync_copy(zbuf_v, acc_sh.at[pl.ds(sid * RPT + k * ZCH, ZCH)])
    plsc.subcore_barrier()

    def _body(i, carry):
        base = wid * EPW + i * CHUNK
        pltpu.sync_copy(src_hbm.at[pl.ds(base, CHUNK)], src_v)
        pltpu.sync_copy(ty_hbm.at[pl.ds(base, CHUNK)], ty_v)
        pltpu.sync_copy(dst_hbm.at[pl.ds(base, CHUNK)], dst_v)
        for j in range(CHUNK // LANES):
            s = pl.ds(j * LANES, LANES)
            comb_v[s] = src_v[s] * T + ty_v[s]
        pltpu.async_copy(z_hbm.at[comb_v], rows_v, sem).wait()
        pltpu.sync_copy(rows_v, acc_sh.at[dst_v], add=True)
        return carry

    lax.fori_loop(0, NCHUNK, _body, 0)
    plsc.subcore_barrier()

    r0 = sid * RPT
    pltpu.sync_copy(acc_sh.at[pl.ds(r0, RPT)], out_hbm.at[cid, pl.ds(r0, RPT)])


@functools.cache
def _sc_accumulate():
    mesh = plsc.VectorSubcoreMesh(
        core_axis_name="c", subcore_axis_name="s",
        num_cores=NC, num_subcores=NS)
    return pl.kernel(
        _sc_accumulate_body,
        out_type=jax.ShapeDtypeStruct((NC, NPAD, H), jnp.float32),
        mesh=mesh,
        scratch_types=[
            pltpu.VMEM((CHUNK,), jnp.int32),
            pltpu.VMEM((CHUNK,), jnp.int32),
            pltpu.VMEM((CHUNK,), jnp.int32),
            pltpu.VMEM((CHUNK,), jnp.int32),
            pltpu.VMEM((CHUNK, H), jnp.float32),
            pltpu.VMEM((ZCH, H), jnp.float32),
            pltpu.VMEM_SHARED((NPAD, H), jnp.float32),
            pltpu.SemaphoreType.DMA,
        ],
    )


def _gru_body(x_ref, p_ref, wi_ref, wh_ref, b_ref, o_ref):
    x = x_ref[...]
    h = p_ref[0] + p_ref[1]
    xg = jnp.dot(x, wi_ref[...], preferred_element_type=jnp.float32) + b_ref[...]
    hg = jnp.dot(h, wh_ref[...], preferred_element_type=jnp.float32)
    r = jax.nn.sigmoid(xg[:, :H] + hg[:, :H])
    z = jax.nn.sigmoid(xg[:, H:2 * H] + hg[:, H:2 * H])
    n = jnp.tanh(xg[:, 2 * H:] + r * hg[:, 2 * H:])
    o_ref[...] = (1.0 - z) * n + z * h


def _gru(x, p, wi, wh, b):
    blk = 1000
    return pl.pallas_call(
        _gru_body,
        grid=(N // blk,),
        in_specs=[
            pl.BlockSpec((blk, H), lambda i: (i, 0)),
            pl.BlockSpec((NC, blk, H), lambda i: (0, i, 0)),
            pl.BlockSpec((H, 3 * H), lambda i: (0, 0)),
            pl.BlockSpec((H, 3 * H), lambda i: (0, 0)),
            pl.BlockSpec((1, 3 * H), lambda i: (0, 0)),
        ],
        out_specs=pl.BlockSpec((blk, H), lambda i: (i, 0)),
        out_shape=jax.ShapeDtypeStruct((N, H), jnp.float32),
    )(x, p, wi, wh, b)


def kernel(statement_embeddings, source_indices, dest_indices, edge_types,
           W_edge, b_edge, Wir, Whr, br, Wiz, Whz, bz, Win, Whn, bn):
    z = _edge_dense(statement_embeddings, W_edge, b_edge.reshape(1, T * H))
    z_rows = z.reshape(N * T, H)
    partials = _sc_accumulate()(z_rows, source_indices, edge_types,
                                dest_indices)
    wi = jnp.concatenate([Wir, Wiz, Win], axis=1)
    wh = jnp.concatenate([Whr, Whz, Whn], axis=1)
    b = jnp.concatenate([br, bz, bn]).reshape(1, 3 * H)
    return _gru(statement_embeddings, partials, wi, wh, b)


# speedup vs baseline: 14.5118x; 2.1957x over previous
import functools

import jax
import jax.numpy as jnp
from jax import lax
from jax.experimental import pallas as pl
from jax.experimental.pallas import tpu as pltpu
from jax.experimental.pallas import tpu_sc as plsc

N = 10000
E = 320000
H = 128
T = 6

NC = 2
NS = 16
NW = NC * NS
EPW = E // NW
CHUNK = 80
NCHUNK = EPW // CHUNK
NPAD = 10240
RPT = NPAD // NS
ZCH = 128
LANES = 16


def _edge_dense_body(x_ref, w_ref, b_ref, z_ref):
    z_ref[...] = (
        jnp.dot(x_ref[...], w_ref[...], preferred_element_type=jnp.float32)
        + b_ref[...]
    )


def _edge_dense(x, w, b):
    blk = 1000
    return pl.pallas_call(
        _edge_dense_body,
        grid=(N // blk,),
        in_specs=[
            pl.BlockSpec((blk, H), lambda i: (i, 0)),
            pl.BlockSpec((H, T * H), lambda i: (0, 0)),
            pl.BlockSpec((1, T * H), lambda i: (0, 0)),
        ],
        out_specs=pl.BlockSpec((blk, T * H), lambda i: (i, 0)),
        out_shape=jax.ShapeDtypeStruct((N, T * H), jnp.float32),
    )(x, w, b)


def _sc_accumulate_body(z_hbm, comb_hbm, dst_hbm, out_hbm,
                        comb1, dst1, rows0, rows1, acc_sh,
                        isem, gsem0, gsem1):
    cid = lax.axis_index("c")
    sid = lax.axis_index("s")
    wid = cid * NS + sid

    c_comb = pltpu.async_copy(comb_hbm.at[pl.ds(wid * EPW, EPW)], comb1, isem)
    c_dst = pltpu.async_copy(dst_hbm.at[pl.ds(wid * EPW, EPW)], dst1, isem)

    def _zrow(r, carry):
        for c in range(H // LANES):
            rows0[r, pl.ds(c * LANES, LANES)] = jnp.zeros((LANES,), jnp.float32)
        return carry

    lax.fori_loop(0, CHUNK, _zrow, 0)
    for k in range(RPT // CHUNK):
        pltpu.sync_copy(rows0, acc_sh.at[pl.ds(sid * RPT + k * CHUNK, CHUNK)])

    c_comb.wait()
    c_dst.wait()
    plsc.subcore_barrier()

    def _cs(i):
        return comb1.at[pl.ds(i * CHUNK, CHUNK)]

    def _ds(i):
        return dst1.at[pl.ds(i * CHUNK, CHUNK)]

    pltpu.async_copy(z_hbm.at[_cs(0)], rows0, gsem0)
    pltpu.async_copy(z_hbm.at[_cs(1)], rows1, gsem1)

    def _pair(k, carry):
        i0 = 2 * k
        pltpu.make_async_copy(z_hbm.at[_cs(i0)], rows0, gsem0).wait()
        pltpu.sync_copy(rows0, acc_sh.at[_ds(i0)], add=True)
        pltpu.async_copy(z_hbm.at[_cs(i0 + 2)], rows0, gsem0)
        pltpu.make_async_copy(z_hbm.at[_cs(i0 + 1)], rows1, gsem1).wait()
        pltpu.sync_copy(rows1, acc_sh.at[_ds(i0 + 1)], add=True)

        @pl.when(i0 + 3 < NCHUNK)
        def _():
            pltpu.async_copy(z_hbm.at[_cs(i0 + 3)], rows1, gsem1)

        return carry

    lax.fori_loop(0, (NCHUNK - 1) // 2, _pair, 0)
    pltpu.make_async_copy(z_hbm.at[_cs(NCHUNK - 1)], rows0, gsem0).wait()
    pltpu.sync_copy(rows0, acc_sh.at[_ds(NCHUNK - 1)], add=True)
    plsc.subcore_barrier()

    r0 = sid * RPT
    pltpu.sync_copy(acc_sh.at[pl.ds(r0, RPT)], out_hbm.at[cid, pl.ds(r0, RPT)])


@functools.cache
def _sc_accumulate():
    mesh = plsc.VectorSubcoreMesh(
        core_axis_name="c", subcore_axis_name="s",
        num_cores=NC, num_subcores=NS)
    return pl.kernel(
        _sc_accumulate_body,
        out_type=jax.ShapeDtypeStruct((NC, NPAD, H), jnp.float32),
        mesh=mesh,
        scratch_types=[
            pltpu.VMEM((EPW,), jnp.int32),
            pltpu.VMEM((EPW,), jnp.int32),
            pltpu.VMEM((CHUNK, H), jnp.float32),
            pltpu.VMEM((CHUNK, H), jnp.float32),
            pltpu.VMEM_SHARED((NPAD, H), jnp.float32),
            pltpu.SemaphoreType.DMA,
            pltpu.SemaphoreType.DMA,
            pltpu.SemaphoreType.DMA,
        ],
    )


def _comb_body(s_ref, t_ref, o_ref):
    o_ref[...] = s_ref[...] * T + t_ref[...]


def _comb(src, ty):
    rows = E // 512
    return pl.pallas_call(
        _comb_body,
        grid=(1,),
        in_specs=[
            pl.BlockSpec((rows, 512), lambda i: (0, 0)),
            pl.BlockSpec((rows, 512), lambda i: (0, 0)),
        ],
        out_specs=pl.BlockSpec((rows, 512), lambda i: (0, 0)),
        out_shape=jax.ShapeDtypeStruct((rows, 512), jnp.int32),
    )(src.reshape(rows, 512), ty.reshape(rows, 512)).reshape(E)


def _gru_body(x_ref, p_ref, wi_ref, wh_ref, b_ref, o_ref):
    x = x_ref[...]
    h = p_ref[0] + p_ref[1]
    xg = jnp.dot(x, wi_ref[...], preferred_element_type=jnp.float32) + b_ref[...]
    hg = jnp.dot(h, wh_ref[...], preferred_element_type=jnp.float32)
    r = jax.nn.sigmoid(xg[:, :H] + hg[:, :H])
    z = jax.nn.sigmoid(xg[:, H:2 * H] + hg[:, H:2 * H])
    n = jnp.tanh(xg[:, 2 * H:] + r * hg[:, 2 * H:])
    o_ref[...] = (1.0 - z) * n + z * h


def _gru(x, p, wi, wh, b):
    blk = 1000
    return pl.pallas_call(
        _gru_body,
        grid=(N // blk,),
        in_specs=[
            pl.BlockSpec((blk, H), lambda i: (i, 0)),
            pl.BlockSpec((NC, blk, H), lambda i: (0, i, 0)),
            pl.BlockSpec((H, 3 * H), lambda i: (0, 0)),
            pl.BlockSpec((H, 3 * H), lambda i: (0, 0)),
            pl.BlockSpec((1, 3 * H), lambda i: (0, 0)),
        ],
        out_specs=pl.BlockSpec((blk, H), lambda i: (i, 0)),
        out_shape=jax.ShapeDtypeStruct((N, H), jnp.float32),
    )(x, p, wi, wh, b)


def kernel(statement_embeddings, source_indices, dest_indices, edge_types,
           W_edge, b_edge, Wir, Whr, br, Wiz, Whz, bz, Win, Whn, bn):
    z = _edge_dense(statement_embeddings, W_edge, b_edge.reshape(1, T * H))
    z_rows = z.reshape(N * T, H)
    comb = _comb(source_indices, edge_types)
    partials = _sc_accumulate()(z_rows, comb, dest_indices)
    wi = jnp.concatenate([Wir, Wiz, Win], axis=1)
    wh = jnp.concatenate([Whr, Whz, Whn], axis=1)
    b = jnp.concatenate([br, bz, bn]).reshape(1, 3 * H)
    return _gru(statement_embeddings, partials, wi, wh, b)


# speedup vs baseline: 14.9170x; 1.0279x over previous
import functools

import jax
import jax.numpy as jnp
from jax import lax
from jax.experimental import pallas as pl
from jax.experimental.pallas import tpu as pltpu
from jax.experimental.pallas import tpu_sc as plsc

N = 10000
E = 320000
H = 128
T = 6

NC = 2
NS = 16
NW = NC * NS
EPW = E // NW
CHUNK = 80
NCHUNK = EPW // CHUNK
NPAD = 10240
RPT = NPAD // NS
ZCH = 128
LANES = 16


def _edge_dense_body(x_ref, w_ref, b_ref, s_ref, t_ref, z_ref, c_ref):
    z_ref[...] = (
        jnp.dot(x_ref[...], w_ref[...], preferred_element_type=jnp.float32)
        + b_ref[...]
    )

    @pl.when(pl.program_id(0) == 0)
    def _():
        c_ref[...] = s_ref[...] * T + t_ref[...]


def _edge_dense(x, w, b, src, ty):
    blk = 1000
    erows = E // 128
    z, comb = pl.pallas_call(
        _edge_dense_body,
        grid=(N // blk,),
        in_specs=[
            pl.BlockSpec((blk, H), lambda i: (i, 0)),
            pl.BlockSpec((H, T * H), lambda i: (0, 0)),
            pl.BlockSpec((1, T * H), lambda i: (0, 0)),
            pl.BlockSpec((erows, 128), lambda i: (0, 0)),
            pl.BlockSpec((erows, 128), lambda i: (0, 0)),
        ],
        out_specs=[
            pl.BlockSpec((blk, T * H), lambda i: (i, 0)),
            pl.BlockSpec((erows, 128), lambda i: (0, 0)),
        ],
        out_shape=[
            jax.ShapeDtypeStruct((N, T * H), jnp.float32),
            jax.ShapeDtypeStruct((E // 128, 128), jnp.int32),
        ],
    )(x, w, b, src.reshape(E // 128, 128), ty.reshape(E // 128, 128))
    return z, comb.reshape(E)


def _sc_accumulate_body(z_hbm, comb_hbm, dst_hbm, out_hbm,
                        comb1, dst1, rows0, rows1, acc_sh,
                        isem, gsem0, gsem1):
    cid = lax.axis_index("c")
    sid = lax.axis_index("s")
    wid = cid * NS + sid

    c_comb = pltpu.async_copy(comb_hbm.at[pl.ds(wid * EPW, EPW)], comb1, isem)
    c_dst = pltpu.async_copy(dst_hbm.at[pl.ds(wid * EPW, EPW)], dst1, isem)

    def _zrow(r, carry):
        for c in range(H // LANES):
            rows0[r, pl.ds(c * LANES, LANES)] = jnp.zeros((LANES,), jnp.float32)
        return carry

    lax.fori_loop(0, CHUNK, _zrow, 0)
    for k in range(RPT // CHUNK):
        pltpu.sync_copy(rows0, acc_sh.at[pl.ds(sid * RPT + k * CHUNK, CHUNK)])

    c_comb.wait()
    c_dst.wait()
    plsc.subcore_barrier()

    def _cs(i):
        return comb1.at[pl.ds(i * CHUNK, CHUNK)]

    def _ds(i):
        return dst1.at[pl.ds(i * CHUNK, CHUNK)]

    pltpu.async_copy(z_hbm.at[_cs(0)], rows0, gsem0)
    pltpu.async_copy(z_hbm.at[_cs(1)], rows1, gsem1)

    def _pair(k, carry):
        i0 = 2 * k
        pltpu.make_async_copy(z_hbm.at[_cs(i0)], rows0, gsem0).wait()
        pltpu.sync_copy(rows0, acc_sh.at[_ds(i0)], add=True)
        pltpu.async_copy(z_hbm.at[_cs(i0 + 2)], rows0, gsem0)
        pltpu.make_async_copy(z_hbm.at[_cs(i0 + 1)], rows1, gsem1).wait()
        pltpu.sync_copy(rows1, acc_sh.at[_ds(i0 + 1)], add=True)

        @pl.when(i0 + 3 < NCHUNK)
        def _():
            pltpu.async_copy(z_hbm.at[_cs(i0 + 3)], rows1, gsem1)

        return carry

    lax.fori_loop(0, (NCHUNK - 1) // 2, _pair, 0)
    pltpu.make_async_copy(z_hbm.at[_cs(NCHUNK - 1)], rows0, gsem0).wait()
    pltpu.sync_copy(rows0, acc_sh.at[_ds(NCHUNK - 1)], add=True)
    plsc.subcore_barrier()

    r0 = sid * RPT
    pltpu.sync_copy(acc_sh.at[pl.ds(r0, RPT)], out_hbm.at[cid, pl.ds(r0, RPT)])


@functools.cache
def _sc_accumulate():
    mesh = plsc.VectorSubcoreMesh(
        core_axis_name="c", subcore_axis_name="s",
        num_cores=NC, num_subcores=NS)
    return pl.kernel(
        _sc_accumulate_body,
        out_type=jax.ShapeDtypeStruct((NC, NPAD, H), jnp.float32),
        mesh=mesh,
        scratch_types=[
            pltpu.VMEM((EPW,), jnp.int32),
            pltpu.VMEM((EPW,), jnp.int32),
            pltpu.VMEM((CHUNK, H), jnp.float32),
            pltpu.VMEM((CHUNK, H), jnp.float32),
            pltpu.VMEM_SHARED((NPAD, H), jnp.float32),
            pltpu.SemaphoreType.DMA,
            pltpu.SemaphoreType.DMA,
            pltpu.SemaphoreType.DMA,
        ],
    )


def _xg_body(x_ref, wi_ref, b_ref, o_ref):
    o_ref[...] = (
        jnp.dot(x_ref[...], wi_ref[...], preferred_element_type=jnp.float32)
        + b_ref[...]
    )


def _xg(x, wi, b):
    blk = 1000
    return pl.pallas_call(
        _xg_body,
        grid=(N // blk,),
        in_specs=[
            pl.BlockSpec((blk, H), lambda i: (i, 0)),
            pl.BlockSpec((H, 3 * H), lambda i: (0, 0)),
            pl.BlockSpec((1, 3 * H), lambda i: (0, 0)),
        ],
        out_specs=pl.BlockSpec((blk, 3 * H), lambda i: (i, 0)),
        out_shape=jax.ShapeDtypeStruct((N, 3 * H), jnp.float32),
    )(x, wi, b)


def _gru_body(xg_ref, p_ref, wh_ref, o_ref):
    h = p_ref[0] + p_ref[1]
    xg = xg_ref[...]
    hg = jnp.dot(h, wh_ref[...], preferred_element_type=jnp.float32)
    r = jax.nn.sigmoid(xg[:, :H] + hg[:, :H])
    z = jax.nn.sigmoid(xg[:, H:2 * H] + hg[:, H:2 * H])
    n = jnp.tanh(xg[:, 2 * H:] + r * hg[:, 2 * H:])
    o_ref[...] = (1.0 - z) * n + z * h


def _gru(xg, p, wh):
    blk = 1000
    return pl.pallas_call(
        _gru_body,
        grid=(N // blk,),
        in_specs=[
            pl.BlockSpec((blk, 3 * H), lambda i: (i, 0)),
            pl.BlockSpec((NC, blk, H), lambda i: (0, i, 0)),
            pl.BlockSpec((H, 3 * H), lambda i: (0, 0)),
        ],
        out_specs=pl.BlockSpec((blk, H), lambda i: (i, 0)),
        out_shape=jax.ShapeDtypeStruct((N, H), jnp.float32),
    )(xg, p, wh)


def kernel(statement_embeddings, source_indices, dest_indices, edge_types,
           W_edge, b_edge, Wir, Whr, br, Wiz, Whz, bz, Win, Whn, bn):
    z, comb = _edge_dense(statement_embeddings, W_edge,
                          b_edge.reshape(1, T * H), source_indices, edge_types)
    z_rows = z.reshape(N * T, H)
    partials = _sc_accumulate()(z_rows, comb, dest_indices)
    wi = jnp.concatenate([Wir, Wiz, Win], axis=1)
    wh = jnp.concatenate([Whr, Whz, Whn], axis=1)
    b = jnp.concatenate([br, bz, bn]).reshape(1, 3 * H)
    xg = _xg(statement_embeddings, wi, b)
    return _gru(xg, partials, wh)


# speedup vs baseline: 15.0606x; 1.0096x over previous
import functools

import jax
import jax.numpy as jnp
from jax import lax
from jax.experimental import pallas as pl
from jax.experimental.pallas import tpu as pltpu
from jax.experimental.pallas import tpu_sc as plsc

N = 10000
E = 320000
H = 128
T = 6

NC = 2
NS = 16
NW = NC * NS
EPW = E // NW
CHUNK = 80
NCHUNK = EPW // CHUNK
NPAD = 10240
RPT = NPAD // NS
ZCH = 128
LANES = 16


def _edge_dense_body(x_ref, w_ref, b_ref, s_ref, t_ref, z_ref, c_ref):
    z_ref[...] = (
        jnp.dot(x_ref[...], w_ref[...], preferred_element_type=jnp.float32)
        + b_ref[...]
    )

    @pl.when((pl.program_id(0) == 0) & (pl.program_id(1) == 0))
    def _():
        c_ref[...] = t_ref[...] * N + s_ref[...]


def _edge_dense(x, w, b, src, ty):
    blk = 1000
    nb = N // blk
    erows = E // 128
    z, comb = pl.pallas_call(
        _edge_dense_body,
        grid=(nb, T),
        in_specs=[
            pl.BlockSpec((blk, H), lambda i, t: (i, 0)),
            pl.BlockSpec((H, H), lambda i, t: (0, t)),
            pl.BlockSpec((1, H), lambda i, t: (0, t)),
            pl.BlockSpec((erows, 128), lambda i, t: (0, 0)),
            pl.BlockSpec((erows, 128), lambda i, t: (0, 0)),
        ],
        out_specs=[
            pl.BlockSpec((blk, H), lambda i, t: (t * nb + i, 0)),
            pl.BlockSpec((erows, 128), lambda i, t: (0, 0)),
        ],
        out_shape=[
            jax.ShapeDtypeStruct((T * N, H), jnp.float32),
            jax.ShapeDtypeStruct((E // 128, 128), jnp.int32),
        ],
    )(x, w, b, src.reshape(E // 128, 128), ty.reshape(E // 128, 128))
    return z, comb.reshape(E)


def _sc_accumulate_body(z_hbm, comb_hbm, dst_hbm, out_hbm,
                        comb1, dst1, rows0, rows1, acc_sh,
                        isem, gsem0, gsem1):
    cid = lax.axis_index("c")
    sid = lax.axis_index("s")
    wid = cid * NS + sid

    c_comb = pltpu.async_copy(comb_hbm.at[pl.ds(wid * EPW, EPW)], comb1, isem)
    c_dst = pltpu.async_copy(dst_hbm.at[pl.ds(wid * EPW, EPW)], dst1, isem)

    def _zrow(r, carry):
        for c in range(H // LANES):
            rows0[r, pl.ds(c * LANES, LANES)] = jnp.zeros((LANES,), jnp.float32)
        return carry

    lax.fori_loop(0, CHUNK, _zrow, 0)
    for k in range(RPT // CHUNK):
        pltpu.sync_copy(rows0, acc_sh.at[pl.ds(sid * RPT + k * CHUNK, CHUNK)])

    c_comb.wait()
    c_dst.wait()
    plsc.subcore_barrier()

    def _cs(i):
        return comb1.at[pl.ds(i * CHUNK, CHUNK)]

    def _ds(i):
        return dst1.at[pl.ds(i * CHUNK, CHUNK)]

    pltpu.async_copy(z_hbm.at[_cs(0)], rows0, gsem0)
    pltpu.async_copy(z_hbm.at[_cs(1)], rows1, gsem1)

    def _pair(k, carry):
        i0 = 2 * k
        pltpu.make_async_copy(z_hbm.at[_cs(i0)], rows0, gsem0).wait()
        pltpu.sync_copy(rows0, acc_sh.at[_ds(i0)], add=True)
        pltpu.async_copy(z_hbm.at[_cs(i0 + 2)], rows0, gsem0)
        pltpu.make_async_copy(z_hbm.at[_cs(i0 + 1)], rows1, gsem1).wait()
        pltpu.sync_copy(rows1, acc_sh.at[_ds(i0 + 1)], add=True)

        @pl.when(i0 + 3 < NCHUNK)
        def _():
            pltpu.async_copy(z_hbm.at[_cs(i0 + 3)], rows1, gsem1)

        return carry

    lax.fori_loop(0, (NCHUNK - 1) // 2, _pair, 0)
    pltpu.make_async_copy(z_hbm.at[_cs(NCHUNK - 1)], rows0, gsem0).wait()
    pltpu.sync_copy(rows0, acc_sh.at[_ds(NCHUNK - 1)], add=True)
    plsc.subcore_barrier()

    r0 = sid * RPT
    pltpu.sync_copy(acc_sh.at[pl.ds(r0, RPT)], out_hbm.at[cid, pl.ds(r0, RPT)])


@functools.cache
def _sc_accumulate():
    mesh = plsc.VectorSubcoreMesh(
        core_axis_name="c", subcore_axis_name="s",
        num_cores=NC, num_subcores=NS)
    return pl.kernel(
        _sc_accumulate_body,
        out_type=jax.ShapeDtypeStruct((NC, NPAD, H), jnp.float32),
        mesh=mesh,
        scratch_types=[
            pltpu.VMEM((EPW,), jnp.int32),
            pltpu.VMEM((EPW,), jnp.int32),
            pltpu.VMEM((CHUNK, H), jnp.float32),
            pltpu.VMEM((CHUNK, H), jnp.float32),
            pltpu.VMEM_SHARED((NPAD, H), jnp.float32),
            pltpu.SemaphoreType.DMA,
            pltpu.SemaphoreType.DMA,
            pltpu.SemaphoreType.DMA,
        ],
    )


def _xg_body(x_ref, wi_ref, b_ref, o_ref):
    o_ref[...] = (
        jnp.dot(x_ref[...], wi_ref[...], preferred_element_type=jnp.float32)
        + b_ref[...]
    )


def _xg(x, wi, b):
    blk = 1000
    return pl.pallas_call(
        _xg_body,
        grid=(N // blk,),
        in_specs=[
            pl.BlockSpec((blk, H), lambda i: (i, 0)),
            pl.BlockSpec((H, 3 * H), lambda i: (0, 0)),
            pl.BlockSpec((1, 3 * H), lambda i: (0, 0)),
        ],
        out_specs=pl.BlockSpec((blk, 3 * H), lambda i: (i, 0)),
        out_shape=jax.ShapeDtypeStruct((N, 3 * H), jnp.float32),
    )(x, wi, b)


def _gru_body(xg_ref, p_ref, wh_ref, o_ref):
    h = p_ref[0] + p_ref[1]
    xg = xg_ref[...]
    hg = jnp.dot(h, wh_ref[...], preferred_element_type=jnp.float32)
    r = jax.nn.sigmoid(xg[:, :H] + hg[:, :H])
    z = jax.nn.sigmoid(xg[:, H:2 * H] + hg[:, H:2 * H])
    n = jnp.tanh(xg[:, 2 * H:] + r * hg[:, 2 * H:])
    o_ref[...] = (1.0 - z) * n + z * h


def _gru(xg, p, wh):
    blk = 1000
    return pl.pallas_call(
        _gru_body,
        grid=(N // blk,),
        in_specs=[
            pl.BlockSpec((blk, 3 * H), lambda i: (i, 0)),
            pl.BlockSpec((NC, blk, H), lambda i: (0, i, 0)),
            pl.BlockSpec((H, 3 * H), lambda i: (0, 0)),
        ],
        out_specs=pl.BlockSpec((blk, H), lambda i: (i, 0)),
        out_shape=jax.ShapeDtypeStruct((N, H), jnp.float32),
    )(xg, p, wh)


def kernel(statement_embeddings, source_indices, dest_indices, edge_types,
           W_edge, b_edge, Wir, Whr, br, Wiz, Whz, bz, Win, Whn, bn):
    z, comb = _edge_dense(statement_embeddings, W_edge,
                          b_edge.reshape(1, T * H), source_indices, edge_types)
    partials = _sc_accumulate()(z, comb, dest_indices)
    wi = jnp.concatenate([Wir, Wiz, Win], axis=1)
    wh = jnp.concatenate([Whr, Whz, Whn], axis=1)
    b = jnp.concatenate([br, bz, bn]).reshape(1, 3 * H)
    xg = _xg(statement_embeddings, wi, b)
    return _gru(xg, partials, wh)


# speedup vs baseline: 17.5300x; 1.1640x over previous
import functools

import jax
import jax.numpy as jnp
from jax import lax
from jax.experimental import pallas as pl
from jax.experimental.pallas import tpu as pltpu
from jax.experimental.pallas import tpu_sc as plsc

N = 10000
E = 320000
H = 128
T = 6

NC = 2
NS = 16
NW = NC * NS
EPW = E // NW
CHUNK = 80
NCHUNK = EPW // CHUNK
NPAD = 10240
RPT = NPAD // NS
ZCH = 128
LANES = 16


def _edge_dense_body(x_ref, w_ref, b_ref, z_ref):
    za = (
        jnp.dot(x_ref[...], w_ref[...], preferred_element_type=jnp.float32)
        + b_ref[...]
    )
    for t in range(T):
        z_ref[t] = za[:, t * H:(t + 1) * H]


def _edge_dense(x, w, b):
    blk = 1000
    return pl.pallas_call(
        _edge_dense_body,
        grid=(N // blk,),
        in_specs=[
            pl.BlockSpec((blk, H), lambda i: (i, 0)),
            pl.BlockSpec((H, T * H), lambda i: (0, 0)),
            pl.BlockSpec((1, T * H), lambda i: (0, 0)),
        ],
        out_specs=pl.BlockSpec((T, blk, H), lambda i: (0, i, 0)),
        out_shape=jax.ShapeDtypeStruct((T, N, H), jnp.float32),
    )(x, w, b)


def _comb_body(s_ref, t_ref, o_ref):
    o_ref[...] = t_ref[...] * N + s_ref[...]


def _comb(src, ty):
    rows = E // 128
    return pl.pallas_call(
        _comb_body,
        grid=(1,),
        in_specs=[
            pl.BlockSpec((rows, 128), lambda i: (0, 0)),
            pl.BlockSpec((rows, 128), lambda i: (0, 0)),
        ],
        out_specs=pl.BlockSpec((rows, 128), lambda i: (0, 0)),
        out_shape=jax.ShapeDtypeStruct((rows, 128), jnp.int32),
    )(src.reshape(rows, 128), ty.reshape(rows, 128)).reshape(E)


def _sc_accumulate_body(z_hbm, comb_hbm, dst_hbm, out_hbm,
                        comb1, dst1, rows0, rows1, acc_sh,
                        isem, gsem0, gsem1):
    cid = lax.axis_index("c")
    sid = lax.axis_index("s")
    wid = cid * NS + sid

    c_comb = pltpu.async_copy(comb_hbm.at[pl.ds(wid * EPW, EPW)], comb1, isem)
    c_dst = pltpu.async_copy(dst_hbm.at[pl.ds(wid * EPW, EPW)], dst1, isem)

    def _zrow(r, carry):
        for c in range(H // LANES):
            rows0[r, pl.ds(c * LANES, LANES)] = jnp.zeros((LANES,), jnp.float32)
        return carry

    lax.fori_loop(0, CHUNK, _zrow, 0)
    for k in range(RPT // CHUNK):
        pltpu.sync_copy(rows0, acc_sh.at[pl.ds(sid * RPT + k * CHUNK, CHUNK)])

    c_comb.wait()
    c_dst.wait()
    plsc.subcore_barrier()

    def _cs(i):
        return comb1.at[pl.ds(i * CHUNK, CHUNK)]

    def _ds(i):
        return dst1.at[pl.ds(i * CHUNK, CHUNK)]

    pltpu.async_copy(z_hbm.at[_cs(0)], rows0, gsem0)
    pltpu.async_copy(z_hbm.at[_cs(1)], rows1, gsem1)

    def _pair(k, carry):
        i0 = 2 * k
        pltpu.make_async_copy(z_hbm.at[_cs(i0)], rows0, gsem0).wait()
        pltpu.sync_copy(rows0, acc_sh.at[_ds(i0)], add=True)
        pltpu.async_copy(z_hbm.at[_cs(i0 + 2)], rows0, gsem0)
        pltpu.make_async_copy(z_hbm.at[_cs(i0 + 1)], rows1, gsem1).wait()
        pltpu.sync_copy(rows1, acc_sh.at[_ds(i0 + 1)], add=True)

        @pl.when(i0 + 3 < NCHUNK)
        def _():
            pltpu.async_copy(z_hbm.at[_cs(i0 + 3)], rows1, gsem1)

        return carry

    lax.fori_loop(0, (NCHUNK - 1) // 2, _pair, 0)
    pltpu.make_async_copy(z_hbm.at[_cs(NCHUNK - 1)], rows0, gsem0).wait()
    pltpu.sync_copy(rows0, acc_sh.at[_ds(NCHUNK - 1)], add=True)
    plsc.subcore_barrier()

    r0 = sid * RPT
    pltpu.sync_copy(acc_sh.at[pl.ds(r0, RPT)], out_hbm.at[cid, pl.ds(r0, RPT)])


@functools.cache
def _sc_accumulate():
    mesh = plsc.VectorSubcoreMesh(
        core_axis_name="c", subcore_axis_name="s",
        num_cores=NC, num_subcores=NS)
    return pl.kernel(
        _sc_accumulate_body,
        out_type=jax.ShapeDtypeStruct((NC, NPAD, H), jnp.float32),
        mesh=mesh,
        scratch_types=[
            pltpu.VMEM((EPW,), jnp.int32),
            pltpu.VMEM((EPW,), jnp.int32),
            pltpu.VMEM((CHUNK, H), jnp.float32),
            pltpu.VMEM((CHUNK, H), jnp.float32),
            pltpu.VMEM_SHARED((NPAD, H), jnp.float32),
            pltpu.SemaphoreType.DMA,
            pltpu.SemaphoreType.DMA,
            pltpu.SemaphoreType.DMA,
        ],
    )


def _xg_body(x_ref, wi_ref, b_ref, o_ref):
    o_ref[...] = (
        jnp.dot(x_ref[...], wi_ref[...], preferred_element_type=jnp.float32)
        + b_ref[...]
    )


def _xg(x, wi, b):
    blk = 1000
    return pl.pallas_call(
        _xg_body,
        grid=(N // blk,),
        in_specs=[
            pl.BlockSpec((blk, H), lambda i: (i, 0)),
            pl.BlockSpec((H, 3 * H), lambda i: (0, 0)),
            pl.BlockSpec((1, 3 * H), lambda i: (0, 0)),
        ],
        out_specs=pl.BlockSpec((blk, 3 * H), lambda i: (i, 0)),
        out_shape=jax.ShapeDtypeStruct((N, 3 * H), jnp.float32),
    )(x, wi, b)


def _gru_body(xg_ref, p_ref, wh_ref, o_ref):
    h = p_ref[0] + p_ref[1]
    xg = xg_ref[...]
    hg = jnp.dot(h, wh_ref[...], preferred_element_type=jnp.float32)
    r = jax.nn.sigmoid(xg[:, :H] + hg[:, :H])
    z = jax.nn.sigmoid(xg[:, H:2 * H] + hg[:, H:2 * H])
    n = jnp.tanh(xg[:, 2 * H:] + r * hg[:, 2 * H:])
    o_ref[...] = (1.0 - z) * n + z * h


def _gru(xg, p, wh):
    blk = 1000
    return pl.pallas_call(
        _gru_body,
        grid=(N // blk,),
        in_specs=[
            pl.BlockSpec((blk, 3 * H), lambda i: (i, 0)),
            pl.BlockSpec((NC, blk, H), lambda i: (0, i, 0)),
            pl.BlockSpec((H, 3 * H), lambda i: (0, 0)),
        ],
        out_specs=pl.BlockSpec((blk, H), lambda i: (i, 0)),
        out_shape=jax.ShapeDtypeStruct((N, H), jnp.float32),
    )(xg, p, wh)


def kernel(statement_embeddings, source_indices, dest_indices, edge_types,
           W_edge, b_edge, Wir, Whr, br, Wiz, Whz, bz, Win, Whn, bn):
    z = _edge_dense(statement_embeddings, W_edge, b_edge.reshape(1, T * H))
    comb = _comb(source_indices, edge_types)
    partials = _sc_accumulate()(z.reshape(T * N, H), comb, dest_indices)
    wi = jnp.concatenate([Wir, Wiz, Win], axis=1)
    wh = jnp.concatenate([Whr, Whz, Whn], axis=1)
    b = jnp.concatenate([br, bz, bn]).reshape(1, 3 * H)
    xg = _xg(statement_embeddings, wi, b)
    return _gru(xg, partials, wh)


# speedup vs baseline: 20.1279x; 1.1482x over previous
import functools

import jax
import jax.numpy as jnp
from jax import lax
from jax.experimental import pallas as pl
from jax.experimental.pallas import tpu as pltpu
from jax.experimental.pallas import tpu_sc as plsc

N = 10000
E = 320000
H = 128
T = 6

NC = 2
NS = 16
NW = NC * NS
EPW = E // NW
CHUNK = 80
NCHUNK = EPW // CHUNK
NPAD = 10240
RPT = NPAD // NS
ZCH = 128
LANES = 16


def _edge_dense_body(x_ref, w_ref, b_ref, z_ref):
    za = (
        jnp.dot(x_ref[...], w_ref[...], preferred_element_type=jnp.float32)
        + b_ref[...]
    )
    for t in range(T):
        z_ref[t] = za[:, t * H:(t + 1) * H]


def _edge_dense(x, w, b):
    blk = 1000
    return pl.pallas_call(
        _edge_dense_body,
        grid=(N // blk,),
        in_specs=[
            pl.BlockSpec((blk, H), lambda i: (i, 0)),
            pl.BlockSpec((H, T * H), lambda i: (0, 0)),
            pl.BlockSpec((1, T * H), lambda i: (0, 0)),
        ],
        out_specs=pl.BlockSpec((T, blk, H), lambda i: (0, i, 0)),
        out_shape=jax.ShapeDtypeStruct((T, N, H), jnp.float32),
    )(x, w, b)


def _comb_body(s_ref, t_ref, o_ref):
    o_ref[...] = t_ref[...] * N + s_ref[...]


def _comb(src, ty):
    rows = E // 128
    return pl.pallas_call(
        _comb_body,
        grid=(1,),
        in_specs=[
            pl.BlockSpec((rows, 128), lambda i: (0, 0)),
            pl.BlockSpec((rows, 128), lambda i: (0, 0)),
        ],
        out_specs=pl.BlockSpec((rows, 128), lambda i: (0, 0)),
        out_shape=jax.ShapeDtypeStruct((rows, 128), jnp.int32),
    )(src.reshape(rows, 128), ty.reshape(rows, 128)).reshape(E)


def _sc_accumulate_body(z_hbm, comb_hbm, dst_hbm, out_hbm,
                        comb1, dring, rows0, rows1, rows2, acc_sh,
                        isem, dsem, gsem0, gsem1, gsem2):
    cid = lax.axis_index("c")
    sid = lax.axis_index("s")
    wid = cid * NS + sid

    c_comb = pltpu.async_copy(comb_hbm.at[pl.ds(wid * EPW, EPW)], comb1, isem)

    def _zrow(r, carry):
        for c in range(H // LANES):
            rows0[r, pl.ds(c * LANES, LANES)] = jnp.zeros((LANES,), jnp.float32)
        return carry

    lax.fori_loop(0, CHUNK, _zrow, 0)
    for k in range(RPT // CHUNK):
        pltpu.sync_copy(rows0, acc_sh.at[pl.ds(sid * RPT + k * CHUNK, CHUNK)])

    c_comb.wait()
    plsc.subcore_barrier()

    def _cs(i):
        return comb1.at[pl.ds(i * CHUNK, CHUNK)]

    def _dload(i, slot):
        return pltpu.async_copy(
            dst_hbm.at[pl.ds(wid * EPW + i * CHUNK, CHUNK)],
            dring.at[slot], dsem)

    bufs = ((rows0, gsem0), (rows1, gsem1), (rows2, gsem2))
    for j in range(3):
        _dload(j, j)
        pltpu.async_copy(z_hbm.at[_cs(j)], bufs[j][0], gsem0 if j == 0
                         else (gsem1 if j == 1 else gsem2))

    def _tri(k, carry):
        i0 = 3 * k
        for j, (rb, gs) in enumerate(bufs):
            i = i0 + j
            pltpu.make_async_copy(
                dst_hbm.at[pl.ds(wid * EPW + i * CHUNK, CHUNK)],
                dring.at[j], dsem).wait()
            pltpu.make_async_copy(z_hbm.at[_cs(i)], rb, gs).wait()
            pltpu.sync_copy(rb, acc_sh.at[dring.at[j]], add=True)

            @pl.when(i + 3 < NCHUNK)
            def _():
                _dload(i + 3, j)
                pltpu.async_copy(z_hbm.at[_cs(i + 3)], rb, gs)

        return carry

    lax.fori_loop(0, NCHUNK // 3, _tri, 0)
    for j in range(NCHUNK - (NCHUNK // 3) * 3):
        i = (NCHUNK // 3) * 3 + j
        pltpu.make_async_copy(
            dst_hbm.at[pl.ds(wid * EPW + i * CHUNK, CHUNK)],
            dring.at[j], dsem).wait()
        pltpu.make_async_copy(z_hbm.at[_cs(i)], bufs[j][0], bufs[j][1]).wait()
        pltpu.sync_copy(bufs[j][0], acc_sh.at[dring.at[j]], add=True)
    plsc.subcore_barrier()

    r0 = sid * RPT
    pltpu.sync_copy(acc_sh.at[pl.ds(r0, RPT)], out_hbm.at[cid, pl.ds(r0, RPT)])


@functools.cache
def _sc_accumulate():
    mesh = plsc.VectorSubcoreMesh(
        core_axis_name="c", subcore_axis_name="s",
        num_cores=NC, num_subcores=NS)
    return pl.kernel(
        _sc_accumulate_body,
        out_type=jax.ShapeDtypeStruct((NC, NPAD, H), jnp.float32),
        mesh=mesh,
        scratch_types=[
            pltpu.VMEM((EPW,), jnp.int32),
            pltpu.VMEM((3, CHUNK), jnp.int32),
            pltpu.VMEM((CHUNK, H), jnp.float32),
            pltpu.VMEM((CHUNK, H), jnp.float32),
            pltpu.VMEM((CHUNK, H), jnp.float32),
            pltpu.VMEM_SHARED((NPAD, H), jnp.float32),
            pltpu.SemaphoreType.DMA,
            pltpu.SemaphoreType.DMA,
            pltpu.SemaphoreType.DMA,
            pltpu.SemaphoreType.DMA,
            pltpu.SemaphoreType.DMA,
        ],
    )


def _xg_body(x_ref, wi_ref, b_ref, o_ref):
    o_ref[...] = (
        jnp.dot(x_ref[...], wi_ref[...], preferred_element_type=jnp.float32)
        + b_ref[...]
    )


def _xg(x, wi, b):
    blk = 1000
    return pl.pallas_call(
        _xg_body,
        grid=(N // blk,),
        in_specs=[
            pl.BlockSpec((blk, H), lambda i: (i, 0)),
            pl.BlockSpec((H, 3 * H), lambda i: (0, 0)),
            pl.BlockSpec((1, 3 * H), lambda i: (0, 0)),
        ],
        out_specs=pl.BlockSpec((blk, 3 * H), lambda i: (i, 0)),
        out_shape=jax.ShapeDtypeStruct((N, 3 * H), jnp.float32),
    )(x, wi, b)


def _gru_body(xg_ref, p_ref, wh_ref, o_ref):
    h = p_ref[0] + p_ref[1]
    xg = xg_ref[...]
    hg = jnp.dot(h, wh_ref[...], preferred_element_type=jnp.float32)
    r = jax.nn.sigmoid(xg[:, :H] + hg[:, :H])
    z = jax.nn.sigmoid(xg[:, H:2 * H] + hg[:, H:2 * H])
    n = jnp.tanh(xg[:, 2 * H:] + r * hg[:, 2 * H:])
    o_ref[...] = (1.0 - z) * n + z * h


def _gru(xg, p, wh):
    blk = 1000
    return pl.pallas_call(
        _gru_body,
        grid=(N // blk,),
        in_specs=[
            pl.BlockSpec((blk, 3 * H), lambda i: (i, 0)),
            pl.BlockSpec((NC, blk, H), lambda i: (0, i, 0)),
            pl.BlockSpec((H, 3 * H), lambda i: (0, 0)),
        ],
        out_specs=pl.BlockSpec((blk, H), lambda i: (i, 0)),
        out_shape=jax.ShapeDtypeStruct((N, H), jnp.float32),
    )(xg, p, wh)


def kernel(statement_embeddings, source_indices, dest_indices, edge_types,
           W_edge, b_edge, Wir, Whr, br, Wiz, Whz, bz, Win, Whn, bn):
    z = _edge_dense(statement_embeddings, W_edge, b_edge.reshape(1, T * H))
    comb = _comb(source_indices, edge_types)
    partials = _sc_accumulate()(z.reshape(T * N, H), comb, dest_indices)
    wi = jnp.concatenate([Wir, Wiz, Win], axis=1)
    wh = jnp.concatenate([Whr, Whz, Whn], axis=1)
    b = jnp.concatenate([br, bz, bn]).reshape(1, 3 * H)
    xg = _xg(statement_embeddings, wi, b)
    return _gru(xg, partials, wh)
